# baseline (device time: 61982 ns/iter reference)
import jax
import jax.numpy as jnp
from jax import lax
from jax.experimental import pallas as pl
from jax.experimental.pallas import tpu as pltpu

N_DEV = 16
B, SQ, SKV = 2, 128, 128
H_LOC, DH = 4, 64
CHUNK = H_LOC * DH
ROWS = B * SQ


def _ag_body(x_ref, out_ref, send_sems, recv_sems):
    my = lax.axis_index("i")
    left = lax.rem(my + N_DEV - 1, N_DEV)
    right = lax.rem(my + 1, N_DEV)

    barrier = pltpu.get_barrier_semaphore()
    pl.semaphore_signal(barrier, inc=1, device_id=(left,),
                        device_id_type=pl.DeviceIdType.MESH)
    pl.semaphore_signal(barrier, inc=1, device_id=(right,),
                        device_id_type=pl.DeviceIdType.MESH)
    pl.semaphore_wait(barrier, 2)

    out_ref[pl.ds(my, 1)] = x_ref[...][None]

    for h in range(N_DEV - 1):
        org = lax.rem(my - h + N_DEV, N_DEV)
        rdma = pltpu.make_async_remote_copy(
            src_ref=out_ref.at[org],
            dst_ref=out_ref.at[org],
            send_sem=send_sems.at[h],
            recv_sem=recv_sems.at[h],
            device_id=(right,),
            device_id_type=pl.DeviceIdType.MESH,
        )
        rdma.start()
        rdma.wait()


def _ring_allgather(ctx2d):
    return pl.pallas_call(
        _ag_body,
        out_shape=jax.ShapeDtypeStruct((N_DEV, ROWS, CHUNK), ctx2d.dtype),
        in_specs=[pl.BlockSpec(memory_space=pltpu.VMEM)],
        out_specs=pl.BlockSpec(memory_space=pltpu.VMEM),
        scratch_shapes=[
            pltpu.SemaphoreType.DMA((N_DEV - 1,)),
            pltpu.SemaphoreType.DMA((N_DEV - 1,)),
        ],
        compiler_params=pltpu.CompilerParams(collective_id=0),
    )(ctx2d)


def kernel(x, Wq, K_ext, V_ext, Wo):
    my = lax.axis_index("i")

    Wq_loc = lax.dynamic_slice(Wq, (0, my * CHUNK), (Wq.shape[0], CHUNK))
    q = (x.astype(jnp.bfloat16) @ Wq_loc.astype(jnp.bfloat16))
    q = q.reshape(B, SQ, H_LOC, DH)

    k = K_ext.astype(jnp.bfloat16)
    v = V_ext.astype(jnp.bfloat16)

    scores = jnp.einsum("bihd,bjhd->bhij", q, k,
                        preferred_element_type=jnp.float32) * 0.125
    qb = (jnp.arange(SQ) // 64)[:, None]
    kb = (jnp.arange(SKV) // 64)[None, :]
    mask = (qb == kb) | ((kb % 4) == (qb % 4))
    scores = jnp.where(mask[None, None], scores, -1e9)
    w = jax.nn.softmax(scores, axis=-1)
    ctx = jnp.einsum("bhij,bjhd->bihd", w.astype(jnp.bfloat16), v,
                     preferred_element_type=jnp.float32)
    ctx2d = ctx.reshape(ROWS, CHUNK).astype(jnp.bfloat16)

    gathered = _ring_allgather(ctx2d)

    wo3 = Wo.reshape(N_DEV, CHUNK, Wo.shape[1]).astype(jnp.bfloat16)
    out = jnp.einsum("krc,kcd->rd", gathered, wo3,
                     preferred_element_type=jnp.float32)
    return out.reshape(B, SQ, Wo.shape[1])


# device time: 29246 ns/iter; 2.1193x vs baseline; 2.1193x over previous
import jax
import jax.numpy as jnp
from jax import lax
from jax.experimental import pallas as pl
from jax.experimental.pallas import tpu as pltpu

N_DEV = 16
B, SQ, SKV = 2, 128, 128
H_LOC, DH = 4, 64
CHUNK = H_LOC * DH
ROWS = B * SQ
D_OUT = 512
N_STEPS = 6


def _ar_body(p_ref, out_ref, send_bufs, recv_bufs, send_sems, recv_sems):
    my = lax.axis_index("i")
    z = my // 4
    q = lax.rem(my, 4)
    by = q // 2
    bx = jnp.bitwise_xor(lax.rem(q, 2), by)

    px = 4 * z + jnp.bitwise_xor(q, 1)
    py = 4 * z + (3 - q)
    pz1 = 4 * jnp.bitwise_xor(z, 1) + q
    pz2 = 4 * jnp.bitwise_xor(z, 2) + q

    barrier = pltpu.get_barrier_semaphore()
    for p in (px, py, pz1, pz2):
        pl.semaphore_signal(barrier, inc=1, device_id=(p,),
                            device_id_type=pl.DeviceIdType.MESH)
    pl.semaphore_wait(barrier, 4)

    out_ref[...] = p_ref[...]

    h0 = bx * 128
    qr0 = h0 + by * 64

    steps = [
        ((1 - bx) * 128,       128, h0,                  True,  px),
        (h0 + (1 - by) * 64,    64, qr0,                 True,  py),
        (qr0,                   64, qr0,                 True,  pz1),
        (qr0,                   64, qr0,                 True,  pz2),
        (qr0,                   64, h0 + (1 - by) * 64,  False, py),
        (h0,                   128, (1 - bx) * 128,      False, px),
    ]
    for s, (src0, n, dst0, is_add, partner) in enumerate(steps):
        send_bufs[s, pl.ds(0, n)] = out_ref[pl.ds(src0, n)].astype(jnp.bfloat16)
        rdma = pltpu.make_async_remote_copy(
            src_ref=send_bufs.at[s, pl.ds(0, n)],
            dst_ref=recv_bufs.at[s, pl.ds(0, n)],
            send_sem=send_sems.at[s],
            recv_sem=recv_sems.at[s],
            device_id=(partner,),
            device_id_type=pl.DeviceIdType.MESH,
        )
        rdma.start()
        rdma.wait()
        r = recv_bufs[s, pl.ds(0, n)].astype(jnp.float32)
        if is_add:
            out_ref[pl.ds(dst0, n)] = out_ref[pl.ds(dst0, n)] + r
        else:
            out_ref[pl.ds(dst0, n)] = r


def _hier_allreduce(partial):
    return pl.pallas_call(
        _ar_body,
        out_shape=jax.ShapeDtypeStruct((ROWS, D_OUT), jnp.float32),
        in_specs=[pl.BlockSpec(memory_space=pltpu.VMEM)],
        out_specs=pl.BlockSpec(memory_space=pltpu.VMEM),
        scratch_shapes=[
            pltpu.VMEM((N_STEPS, 128, D_OUT), jnp.bfloat16),
            pltpu.VMEM((N_STEPS, 128, D_OUT), jnp.bfloat16),
            pltpu.SemaphoreType.DMA((N_STEPS,)),
            pltpu.SemaphoreType.DMA((N_STEPS,)),
        ],
        compiler_params=pltpu.CompilerParams(collective_id=0),
    )(partial)


def kernel(x, Wq, K_ext, V_ext, Wo):
    my = lax.axis_index("i")

    Wq_loc = lax.dynamic_slice(Wq, (0, my * CHUNK), (Wq.shape[0], CHUNK))
    q = (x.astype(jnp.bfloat16) @ Wq_loc.astype(jnp.bfloat16))
    q = q.reshape(B, SQ, H_LOC, DH)

    k = K_ext.astype(jnp.bfloat16)
    v = V_ext.astype(jnp.bfloat16)

    scores = jnp.einsum("bihd,bjhd->bhij", q, k,
                        preferred_element_type=jnp.float32) * 0.125
    qb = (jnp.arange(SQ) // 64)[:, None]
    kb = (jnp.arange(SKV) // 64)[None, :]
    mask = (qb == kb) | ((kb % 4) == (qb % 4))
    scores = jnp.where(mask[None, None], scores, -1e9)
    w = jax.nn.softmax(scores, axis=-1)
    ctx = jnp.einsum("bhij,bjhd->bihd", w.astype(jnp.bfloat16), v,
                     preferred_element_type=jnp.float32)
    ctx2d = ctx.reshape(ROWS, CHUNK).astype(jnp.bfloat16)

    Wo_loc = lax.dynamic_slice(Wo, (my * CHUNK, 0), (CHUNK, Wo.shape[1]))
    partial = jnp.matmul(ctx2d, Wo_loc.astype(jnp.bfloat16),
                         preferred_element_type=jnp.float32)

    out = _hier_allreduce(partial)
    return out.reshape(B, SQ, D_OUT)


# device time: 20791 ns/iter; 2.9812x vs baseline; 1.4067x over previous
import jax
import jax.numpy as jnp
from jax import lax
from jax.experimental import pallas as pl
from jax.experimental.pallas import tpu as pltpu

N_DEV = 16
B, SQ, SKV = 2, 128, 128
H_LOC, DH = 4, 64
CHUNK = H_LOC * DH
ROWS = B * SQ
D_OUT = 512
PIECE = ROWS // N_DEV


def _ar_body(p_ref, out_ref, stage, rs_buf, ag_buf,
             rs_send_sems, rs_recv_sems, ag_send_sems, ag_recv_sems):
    my = lax.axis_index("i")

    barrier = pltpu.get_barrier_semaphore()
    for k in range(1, N_DEV):
        pl.semaphore_signal(barrier, inc=1,
                            device_id=(lax.rem(my + k, N_DEV),),
                            device_id_type=pl.DeviceIdType.MESH)

    stage[...] = p_ref[...].astype(jnp.bfloat16).reshape(N_DEV, PIECE, D_OUT)
    rs_buf[pl.ds(my, 1)] = stage[pl.ds(my, 1)]

    pl.semaphore_wait(barrier, N_DEV - 1)

    rs_rdmas = []
    for k in range(1, N_DEV):
        j = lax.rem(my + k, N_DEV)
        rdma = pltpu.make_async_remote_copy(
            src_ref=stage.at[j],
            dst_ref=rs_buf.at[my],
            send_sem=rs_send_sems.at[j],
            recv_sem=rs_recv_sems.at[my],
            device_id=(j,),
            device_id_type=pl.DeviceIdType.MESH,
        )
        rdma.start()
        rs_rdmas.append(rdma)
    for k in range(1, N_DEV):
        s = lax.rem(my + k, N_DEV)
        recv = pltpu.make_async_remote_copy(
            src_ref=stage.at[s],
            dst_ref=rs_buf.at[s],
            send_sem=rs_send_sems.at[s],
            recv_sem=rs_recv_sems.at[s],
            device_id=(s,),
            device_id_type=pl.DeviceIdType.MESH,
        )
        recv.wait_recv()

    piece = jnp.sum(rs_buf[...].astype(jnp.float32), axis=0)

    ag_buf[pl.ds(my, 1)] = piece.astype(jnp.bfloat16)[None]
    ag_rdmas = []
    for k in range(1, N_DEV):
        j = lax.rem(my + k, N_DEV)
        rdma = pltpu.make_async_remote_copy(
            src_ref=ag_buf.at[my],
            dst_ref=ag_buf.at[my],
            send_sem=ag_send_sems.at[j],
            recv_sem=ag_recv_sems.at[my],
            device_id=(j,),
            device_id_type=pl.DeviceIdType.MESH,
        )
        rdma.start()
        ag_rdmas.append(rdma)
    for k in range(1, N_DEV):
        s = lax.rem(my + k, N_DEV)
        recv = pltpu.make_async_remote_copy(
            src_ref=ag_buf.at[s],
            dst_ref=ag_buf.at[s],
            send_sem=ag_send_sems.at[s],
            recv_sem=ag_recv_sems.at[s],
            device_id=(s,),
            device_id_type=pl.DeviceIdType.MESH,
        )
        recv.wait_recv()

    out_ref[...] = ag_buf[...].astype(jnp.float32).reshape(ROWS, D_OUT)

    for rdma in rs_rdmas + ag_rdmas:
        rdma.wait_send()


def _a2a_allreduce(partial):
    return pl.pallas_call(
        _ar_body,
        out_shape=jax.ShapeDtypeStruct((ROWS, D_OUT), jnp.float32),
        in_specs=[pl.BlockSpec(memory_space=pltpu.VMEM)],
        out_specs=pl.BlockSpec(memory_space=pltpu.VMEM),
        scratch_shapes=[
            pltpu.VMEM((N_DEV, PIECE, D_OUT), jnp.bfloat16),
            pltpu.VMEM((N_DEV, PIECE, D_OUT), jnp.bfloat16),
            pltpu.VMEM((N_DEV, PIECE, D_OUT), jnp.bfloat16),
            pltpu.SemaphoreType.DMA((N_DEV,)),
            pltpu.SemaphoreType.DMA((N_DEV,)),
            pltpu.SemaphoreType.DMA((N_DEV,)),
            pltpu.SemaphoreType.DMA((N_DEV,)),
        ],
        compiler_params=pltpu.CompilerParams(collective_id=0),
    )(partial)


def kernel(x, Wq, K_ext, V_ext, Wo):
    my = lax.axis_index("i")

    Wq_loc = lax.dynamic_slice(Wq, (0, my * CHUNK), (Wq.shape[0], CHUNK))
    q = (x.astype(jnp.bfloat16) @ Wq_loc.astype(jnp.bfloat16))
    q = q.reshape(B, SQ, H_LOC, DH)

    k = K_ext.astype(jnp.bfloat16)
    v = V_ext.astype(jnp.bfloat16)

    scores = jnp.einsum("bihd,bjhd->bhij", q, k,
                        preferred_element_type=jnp.float32) * 0.125
    qb = (jnp.arange(SQ) // 64)[:, None]
    kb = (jnp.arange(SKV) // 64)[None, :]
    mask = (qb == kb) | ((kb % 4) == (qb % 4))
    scores = jnp.where(mask[None, None], scores, -1e9)
    w = jax.nn.softmax(scores, axis=-1)
    ctx = jnp.einsum("bhij,bjhd->bihd", w.astype(jnp.bfloat16), v,
                     preferred_element_type=jnp.float32)
    ctx2d = ctx.reshape(ROWS, CHUNK).astype(jnp.bfloat16)

    Wo_loc = lax.dynamic_slice(Wo, (my * CHUNK, 0), (CHUNK, Wo.shape[1]))
    partial = jnp.matmul(ctx2d, Wo_loc.astype(jnp.bfloat16),
                         preferred_element_type=jnp.float32)

    out = _a2a_allreduce(partial)
    return out.reshape(B, SQ, D_OUT)
